# baseline (speedup 1.0000x reference)
import jax
import jax.numpy as jnp
from jax import lax
from jax.experimental import pallas as pl
from jax.experimental.pallas import tpu as pltpu
from jax.experimental.pallas import tpu_sc as plsc

N = 100000
K = 27
M = 20000
C = 128
E = K * M

NC, NS, L = 2, 16, 16
NW = NC * NS

KA = 14
EA = KA * M
EB = E - EA
SA = 294912
GCH = 128
GSET = 3
SCH = 64

R = 12544
WIN = 8
WPS = WIN // NC
N_PAD = R * WIN
RPT = R // NS
EPS = SA // NS
OC = 1536
NOC = EPS // OC
BT = 8
NSB = 3
RING = 4 * SCH
SENTINEL = 1 << 30

_MESH = plsc.VectorSubcoreMesh(core_axis_name="c", subcore_axis_name="s")
_PARAMS = pltpu.CompilerParams(needs_layout_passes=False)


def _gather_body(feat_hbm, idx_hbm, out_hbm, idx_v,
                 ra0, ra1, ra2, rb0, rb1, rb2,
                 sem_ga, sem_gb, sem_wa, sem_wb):
    epw = idx_hbm.shape[0] // NW
    ngrp = epw // (GCH * GSET)
    wid = lax.axis_index("s") * NC + lax.axis_index("c")
    base = wid * epw
    pltpu.sync_copy(idx_hbm.at[pl.ds(base, epw)], idx_v)

    sets = ((ra0, ra1, ra2, sem_ga, sem_wa), (rb0, rb1, rb2, sem_gb, sem_wb))

    def start_g(g, s):
        r0, r1, r2, sg, _ = sets[s]
        for b, rr in enumerate((r0, r1, r2)):
            ch = g * GSET + b
            pltpu.async_copy(
                feat_hbm.at[idx_v.at[pl.ds(ch * GCH, GCH)]], rr, sg)

    def wait_g(s):
        r0, r1, r2, sg, _ = sets[s]
        for rr in (r0, r1, r2):
            pltpu.make_async_copy(feat_hbm.at[idx_v.at[pl.ds(0, GCH)]],
                                  rr, sg).wait()

    def start_w(g, s):
        r0, r1, r2, _, sw = sets[s]
        for b, rr in enumerate((r0, r1, r2)):
            ch = g * GSET + b
            pltpu.async_copy(rr, out_hbm.at[pl.ds(base + ch * GCH, GCH)], sw)

    def wait_w(s):
        r0, r1, r2, _, sw = sets[s]
        for rr in (r0, r1, r2):
            pltpu.make_async_copy(rr, out_hbm.at[pl.ds(base, GCH)], sw).wait()

    def pair(t, carry):
        for s in range(2):
            g = 2 * t + s

            @pl.when(g >= 2)
            def _():
                wait_w(s)
            start_g(g, s)

            @pl.when(g >= 1)
            def _():
                wait_g(1 - s)
                start_w(g - 1, 1 - s)
        return carry

    lax.fori_loop(0, ngrp // 2, pair, 0)
    wait_g(1)
    start_w(ngrp - 1, 1)
    wait_w(0)
    wait_w(1)


def _gather(features, idx_pad):
    epw = idx_pad.shape[0] // NW
    f = pl.kernel(
        _gather_body,
        out_type=jax.ShapeDtypeStruct((idx_pad.shape[0], C), jnp.float32),
        mesh=_MESH,
        compiler_params=_PARAMS,
        scratch_types=[pltpu.VMEM((epw,), jnp.int32)]
        + [pltpu.VMEM((GCH, C), jnp.float32) for _ in range(6)]
        + [pltpu.SemaphoreType.DMA for _ in range(4)],
    )
    return f(features, idx_pad)


def _gemm_block(g_ref, w_ref, o_ref):
    o_ref[...] = jnp.dot(g_ref[...], w_ref[0],
                         preferred_element_type=jnp.float32)


def _gemm(gathered, weight):
    BM = 2000
    ko = weight.shape[0]
    grid = (ko, M // BM)
    return pl.pallas_call(
        _gemm_block,
        grid=grid,
        in_specs=[
            pl.BlockSpec((BM, C), lambda k, mb: (k * (M // BM) + mb, 0)),
            pl.BlockSpec((1, C, C), lambda k, mb: (k, 0, 0)),
        ],
        out_specs=pl.BlockSpec((BM, C), lambda k, mb: (k * (M // BM) + mb, 0)),
        out_shape=jax.ShapeDtypeStruct((SA, C), jnp.float32),
    )(gathered, weight)


def _scatter_body(ca_hbm, cb_hbm, omap_hbm, bias_hbm, out_hbm,
                  omap_c0, omap_c1, sel_v, bias_v, bias_t,
                  pos0, row0, rows0, pos1, row1, rows1, pos2, row2, rows2,
                  acc_sh, sem_i, sem_o,
                  sem_g0, sem_g1, sem_g2, sem_s0, sem_s1, sem_s2):
    cid = lax.axis_index("c")
    sid = lax.axis_index("s")
    ebase = sid * EPS

    pltpu.sync_copy(bias_hbm, bias_v)
    for r in range(BT):
        for j in range(C // L):
            bias_t[r, pl.ds(j * L, L)] = bias_v[pl.ds(j * L, L)]

    lane = lax.iota(jnp.int32, L)
    lane14 = lane << jnp.full((L,), 14, jnp.int32)
    maskring = jnp.full((L,), RING - 1, jnp.int32)
    ebase_v = jnp.full((L,), ebase, jnp.int32)
    fourteen = jnp.full((L,), 14, jnp.int32)
    mask14 = jnp.full((L,), 16383, jnp.int32)
    dummy = jnp.full((L,), R, jnp.int32)
    ones = jnp.full((L,), 1, jnp.int32)

    omaps = (omap_c0, omap_c1)
    bufs = ((pos0, row0, rows0, sem_g0, sem_s0),
            (pos1, row1, rows1, sem_g1, sem_s1),
            (pos2, row2, rows2, sem_g2, sem_s2))

    def phase_scan(contrib_hbm, obase, lo_v, hi_v):

        def unpack_start(k, b):
            pos_c, row_c, rows_v, sg, _ = bufs[b]
            boff = (k & 3) * SCH
            for j in range(SCH // L):
                pk = sel_v[pl.ds(boff + j * L, L)]
                pos_c[pl.ds(j * L, L)] = (pk >> fourteen) + ebase_v
                row_c[pl.ds(j * L, L)] = pk & mask14
            pltpu.async_copy(contrib_hbm.at[pos_c], rows_v, sg)

        def wait_g(b):
            pos_c, _, rows_v, sg, _ = bufs[b]
            pltpu.make_async_copy(contrib_hbm.at[pos_c], rows_v, sg).wait()

        def start_s(b):
            _, row_c, rows_v, _, ss = bufs[b]
            pltpu.async_copy(rows_v, acc_sh.at[row_c], ss, add=True)

        def wait_s(b):
            _, row_c, rows_v, _, ss = bufs[b]
            pltpu.make_async_copy(rows_v, acc_sh.at[row_c], ss).wait()

        def flush(cnt):
            k = cnt >> 6
            kb = lax.rem(k, NSB)
            for b in range(NSB):
                @pl.when(kb == b)
                def _():
                    @pl.when(k > 0)
                    def _():
                        wait_g((b + NSB - 1) % NSB)
                        start_s((b + NSB - 1) % NSB)

                    @pl.when(k > 2)
                    def _():
                        wait_s(b)
                    unpack_start(k, b)

        def append(cnt, packed, m):
            mi = m.astype(jnp.int32)
            ps = plsc.cumsum(mi)
            cnt_v = jnp.full((L,), cnt - 1, jnp.int32)
            plsc.store_scatter(sel_v, [(cnt_v + ps) & maskring], packed,
                               mask=m)
            newcnt = cnt + ps[15]

            @pl.when((newcnt >> 6) > (cnt >> 6))
            def _():
                flush(cnt)
            return newcnt

        def load_omap(ci, which):
            pltpu.async_copy(omap_hbm.at[pl.ds(obase + ebase + ci * OC, OC)],
                             omaps[which], sem_o)

        def wait_omap(which):
            pltpu.make_async_copy(omap_hbm.at[pl.ds(0, OC)],
                                  omaps[which], sem_o).wait()

        load_omap(0, 0)

        def scan_chunk(ci, cnt, which):
            wait_omap(which)

            @pl.when(ci + 1 < NOC)
            def _():
                load_omap(ci + 1, 1 - which)
            omap_c = omaps[which]

            def scan(i, cnt):
                v = omap_c[pl.ds(i * L, L)]
                m = (v >= lo_v) & (v < hi_v)
                base_v = jnp.full((L,), ci * OC, jnp.int32) + \
                    jnp.full((L,), i * L, jnp.int32)
                packed = ((base_v << fourteen) + lane14) + (v - lo_v)
                return append(cnt, packed, m)
            return lax.fori_loop(0, OC // L, scan, cnt)

        def chunk_pair(t, cnt):
            cnt = scan_chunk(2 * t, cnt, 0)
            cnt = scan_chunk(2 * t + 1, cnt, 1)
            return cnt
        cnt = lax.fori_loop(0, NOC // 2, chunk_pair, 0)

        def pad(i, cnt):
            return append(cnt, dummy, ones > 0)
        cntf = lax.fori_loop(0, SCH // L, pad, cnt)

        kf = (cntf >> 6) - 1
        kfb = lax.rem(kf, NSB)
        for b in range(NSB):
            @pl.when(kfb == b)
            def _():
                wait_g(b)
                start_s(b)
                wait_s(b)

                @pl.when(kf > 0)
                def _():
                    wait_s((b + NSB - 1) % NSB)

                @pl.when(kf > 1)
                def _():
                    wait_s((b + NSB - 2) % NSB)

    def win_pass(p, carry):
        win = cid * WPS + p
        lo = win * R
        lo_v = jnp.full((L,), lo, jnp.int32)
        hi_v = jnp.full((L,), lo + R, jnp.int32)

        def init(i, c):
            pltpu.async_copy(bias_t, acc_sh.at[pl.ds(sid * RPT + i * BT, BT)],
                             sem_i)
            return c
        lax.fori_loop(0, RPT // BT, init, 0)

        def init_w(i, c):
            pltpu.make_async_copy(bias_t, acc_sh.at[pl.ds(0, BT)],
                                  sem_i).wait()
            return c
        lax.fori_loop(0, RPT // BT, init_w, 0)
        plsc.subcore_barrier()

        phase_scan(ca_hbm, 0, lo_v, hi_v)
        phase_scan(cb_hbm, SA, lo_v, hi_v)
        plsc.subcore_barrier()

        pltpu.sync_copy(acc_sh.at[pl.ds(sid * RPT, RPT)],
                        out_hbm.at[pl.ds(win * R + sid * RPT, RPT)])
        plsc.subcore_barrier()
        return carry

    lax.fori_loop(0, WPS, win_pass, 0)


def _scatter(contrib_a, contrib_b, omap_all, bias):
    f = pl.kernel(
        _scatter_body,
        out_type=jax.ShapeDtypeStruct((N_PAD, C), jnp.float32),
        mesh=_MESH,
        compiler_params=_PARAMS,
        scratch_types=[
            pltpu.VMEM((OC,), jnp.int32),
            pltpu.VMEM((OC,), jnp.int32),
            pltpu.VMEM((RING,), jnp.int32),
            pltpu.VMEM((C,), jnp.float32),
            pltpu.VMEM((BT, C), jnp.float32),
            pltpu.VMEM((SCH,), jnp.int32),
            pltpu.VMEM((SCH,), jnp.int32),
            pltpu.VMEM((SCH, C), jnp.float32),
            pltpu.VMEM((SCH,), jnp.int32),
            pltpu.VMEM((SCH,), jnp.int32),
            pltpu.VMEM((SCH, C), jnp.float32),
            pltpu.VMEM((SCH,), jnp.int32),
            pltpu.VMEM((SCH,), jnp.int32),
            pltpu.VMEM((SCH, C), jnp.float32),
            pltpu.VMEM_SHARED((R + L, C), jnp.float32),
        ] + [pltpu.SemaphoreType.DMA for _ in range(8)],
    )
    return f(contrib_a, contrib_b, omap_all, bias)


def kernel(features, in_map, out_map, weight, bias):
    im = in_map.reshape(-1)
    om = out_map.reshape(-1)
    ima = jnp.concatenate([im[:EA], jnp.zeros((SA - EA,), jnp.int32)])
    imb = jnp.concatenate([im[EA:], jnp.zeros((SA - EB,), jnp.int32)])
    oma = jnp.concatenate(
        [om[:EA], jnp.full((SA - EA,), SENTINEL, jnp.int32)])
    omb = jnp.concatenate(
        [om[EA:], jnp.full((SA - EB,), SENTINEL, jnp.int32)])
    om_all = jnp.concatenate([oma, omb])
    ga = _gather(features, ima)
    gb = _gather(features, imb)
    ca = _gemm(ga, weight[:KA])
    cb = _gemm(gb, weight[KA:])
    out_pad = _scatter(ca, cb, om_all, bias)
    return out_pad[:N]


# speedup vs baseline: 3.4746x; 3.4746x over previous
import jax
import jax.numpy as jnp
from jax import lax
from jax.experimental import pallas as pl
from jax.experimental.pallas import tpu as pltpu
from jax.experimental.pallas import tpu_sc as plsc

N = 100000
K = 27
M = 20000
C = 128
CI = 64
E = K * M

NC, NS, L = 2, 16, 16
NW = NC * NS

GCH = 128
EPW = 16896
E_PAD = EPW * NW
EPS = E_PAD // NS
GSET = 3
NGRP = EPW // (GCH * GSET)
SCH = 64

R = 12544
WIN = 8
WPS = WIN // NC
N_PAD = R * WIN
RPT = R // NS
OC = 1536
NOC = EPS // OC
BT = 8
NSB = 3
RING = 4 * SCH
SENTINEL = 1 << 30

_MESH = plsc.VectorSubcoreMesh(core_axis_name="c", subcore_axis_name="s")
_PARAMS = pltpu.CompilerParams(needs_layout_passes=False)
_PARAMS_NT = pltpu.CompilerParams(needs_layout_passes=False,
                                  use_tc_tiling_on_sc=False)


def _gather_body(feat_hbm, idx_hbm, out_hbm, idx_v,
                 ra0, ra1, ra2, rb0, rb1, rb2,
                 sem_ga, sem_gb, sem_wa, sem_wb):
    wid = lax.axis_index("s") * NC + lax.axis_index("c")
    base = wid * EPW
    pltpu.sync_copy(idx_hbm.at[pl.ds(base, EPW)], idx_v)

    sets = ((ra0, ra1, ra2, sem_ga, sem_wa), (rb0, rb1, rb2, sem_gb, sem_wb))

    def start_g(g, s):
        r0, r1, r2, sg, _ = sets[s]
        for b, rr in enumerate((r0, r1, r2)):
            ch = g * GSET + b
            pltpu.async_copy(
                feat_hbm.at[idx_v.at[pl.ds(ch * GCH, GCH)]], rr, sg)

    def wait_g(s):
        r0, r1, r2, sg, _ = sets[s]
        for rr in (r0, r1, r2):
            pltpu.make_async_copy(feat_hbm.at[idx_v.at[pl.ds(0, GCH)]],
                                  rr, sg).wait()

    def start_w(g, s):
        r0, r1, r2, _, sw = sets[s]
        for b, rr in enumerate((r0, r1, r2)):
            ch = g * GSET + b
            pltpu.async_copy(rr, out_hbm.at[pl.ds(base + ch * GCH, GCH)], sw)

    def wait_w(s):
        r0, r1, r2, _, sw = sets[s]
        for rr in (r0, r1, r2):
            pltpu.make_async_copy(rr, out_hbm.at[pl.ds(base, GCH)], sw).wait()

    def pair(t, carry):
        for s in range(2):
            g = 2 * t + s

            @pl.when(g >= 2)
            def _():
                wait_w(s)
            start_g(g, s)

            @pl.when(g >= 1)
            def _():
                wait_g(1 - s)
                start_w(g - 1, 1 - s)
        return carry

    lax.fori_loop(0, NGRP // 2, pair, 0)
    wait_g(1)
    start_w(NGRP - 1, 1)
    wait_w(0)
    wait_w(1)


def _gather(features, idx_pad):
    f = pl.kernel(
        _gather_body,
        out_type=jax.ShapeDtypeStruct((E_PAD, C), jnp.float32),
        mesh=_MESH,
        compiler_params=_PARAMS,
        scratch_types=[pltpu.VMEM((EPW,), jnp.int32)]
        + [pltpu.VMEM((GCH, C), jnp.float32) for _ in range(6)]
        + [pltpu.SemaphoreType.DMA for _ in range(4)],
    )
    return f(features, idx_pad)


def _gemm_block(g_ref, w_ref, o_ref):
    o_ref[...] = jnp.dot(g_ref[...], w_ref[0],
                         preferred_element_type=jnp.float32)


def _gemm(gathered, weight):
    BM = 4000
    grid = (K, M // BM)
    return pl.pallas_call(
        _gemm_block,
        grid=grid,
        in_specs=[
            pl.BlockSpec((BM, C), lambda k, mb: (k * (M // BM) + mb, 0)),
            pl.BlockSpec((1, C, C), lambda k, mb: (k, 0, 0)),
        ],
        out_specs=pl.BlockSpec((BM, C), lambda k, mb: (k * (M // BM) + mb, 0)),
        out_shape=jax.ShapeDtypeStruct((E, C), jnp.float32),
    )(gathered, weight)


def _scatter_body(contrib_hbm, omap_hbm, bias_hbm, out_hbm,
                  omap_c0, omap_c1, sel_v, bias_v, bias_t,
                  pos0, row0, rows0, pos1, row1, rows1, pos2, row2, rows2,
                  acc_sh, sem_i, sem_o,
                  sem_g0, sem_g1, sem_g2, sem_s0, sem_s1, sem_s2):
    cid = lax.axis_index("c")
    sid = lax.axis_index("s")
    ebase = sid * EPS

    pltpu.sync_copy(bias_hbm, bias_v)
    for r in range(BT):
        for j in range(C // L):
            bias_t[r, pl.ds(j * L, L)] = bias_v[pl.ds(j * L, L)]

    lane = lax.iota(jnp.int32, L)
    lane14 = lane << jnp.full((L,), 14, jnp.int32)
    maskring = jnp.full((L,), RING - 1, jnp.int32)
    ebase_v = jnp.full((L,), ebase, jnp.int32)
    fourteen = jnp.full((L,), 14, jnp.int32)
    mask14 = jnp.full((L,), 16383, jnp.int32)
    dummy = jnp.full((L,), R, jnp.int32)
    ones = jnp.full((L,), 1, jnp.int32)

    omaps = (omap_c0, omap_c1)
    bufs = ((pos0, row0, rows0, sem_g0, sem_s0),
            (pos1, row1, rows1, sem_g1, sem_s1),
            (pos2, row2, rows2, sem_g2, sem_s2))

    def unpack_start(k, b):
        pos_c, row_c, rows_v, sg, _ = bufs[b]
        boff = (k & 3) * SCH
        for j in range(SCH // L):
            pk = sel_v[pl.ds(boff + j * L, L)]
            pos_c[pl.ds(j * L, L)] = (pk >> fourteen) + ebase_v
            row_c[pl.ds(j * L, L)] = pk & mask14
        pltpu.async_copy(contrib_hbm.at[pos_c], rows_v, sg)

    def wait_g(b):
        pos_c, _, rows_v, sg, _ = bufs[b]
        pltpu.make_async_copy(contrib_hbm.at[pos_c], rows_v, sg).wait()

    def start_s(b):
        _, row_c, rows_v, _, ss = bufs[b]
        pltpu.async_copy(rows_v, acc_sh.at[row_c], ss, add=True)

    def wait_s(b):
        _, row_c, rows_v, _, ss = bufs[b]
        pltpu.make_async_copy(rows_v, acc_sh.at[row_c], ss).wait()

    def flush(cnt):
        k = cnt >> 6
        kb = lax.rem(k, NSB)
        for b in range(NSB):
            @pl.when(kb == b)
            def _():
                @pl.when(k > 0)
                def _():
                    wait_g((b + NSB - 1) % NSB)
                    start_s((b + NSB - 1) % NSB)
                @pl.when(k > 2)
                def _():
                    wait_s(b)
                unpack_start(k, b)

    def append(cnt, packed, m):
        mi = m.astype(jnp.int32)
        ps = plsc.cumsum(mi)
        cnt_v = jnp.full((L,), cnt - 1, jnp.int32)
        plsc.store_scatter(sel_v, [(cnt_v + ps) & maskring], packed, mask=m)
        newcnt = cnt + ps[15]

        @pl.when((newcnt >> 6) > (cnt >> 6))
        def _():
            flush(cnt)
        return newcnt

    def load_omap(ci, which):
        pltpu.async_copy(omap_hbm.at[pl.ds(ebase + ci * OC, OC)],
                         omaps[which], sem_o)

    def wait_omap(which):
        pltpu.make_async_copy(omap_hbm.at[pl.ds(0, OC)],
                              omaps[which], sem_o).wait()

    def win_pass(p, carry):
        win = cid * WPS + p
        lo = win * R
        lo_v = jnp.full((L,), lo, jnp.int32)
        hi_v = jnp.full((L,), lo + R, jnp.int32)

        def init(i, c):
            pltpu.async_copy(bias_t, acc_sh.at[pl.ds(sid * RPT + i * BT, BT)],
                             sem_i)
            return c
        lax.fori_loop(0, RPT // BT, init, 0)
        load_omap(0, 0)

        def init_w(i, c):
            pltpu.make_async_copy(bias_t, acc_sh.at[pl.ds(0, BT)],
                                  sem_i).wait()
            return c
        lax.fori_loop(0, RPT // BT, init_w, 0)
        plsc.subcore_barrier()

        def scan_chunk(ci, cnt, which):
            wait_omap(which)

            @pl.when(ci + 1 < NOC)
            def _():
                load_omap(ci + 1, 1 - which)
            omap_c = omaps[which]

            def scan(i, cnt):
                v = omap_c[pl.ds(i * L, L)]
                m = (v >= lo_v) & (v < hi_v)
                base_v = jnp.full((L,), ci * OC, jnp.int32) + \
                    jnp.full((L,), i * L, jnp.int32)
                packed = ((base_v << fourteen) + lane14) + (v - lo_v)
                return append(cnt, packed, m)
            return lax.fori_loop(0, OC // L, scan, cnt)

        def chunk_pair(t, cnt):
            cnt = scan_chunk(2 * t, cnt, 0)
            cnt = scan_chunk(2 * t + 1, cnt, 1)
            return cnt
        cnt = lax.fori_loop(0, NOC // 2, chunk_pair, 0)

        def pad(i, cnt):
            return append(cnt, dummy, ones > 0)
        cntf = lax.fori_loop(0, SCH // L, pad, cnt)

        kf = (cntf >> 6) - 1
        kfb = lax.rem(kf, NSB)
        for b in range(NSB):
            @pl.when(kfb == b)
            def _():
                wait_g(b)
                start_s(b)
                wait_s(b)

                @pl.when(kf > 0)
                def _():
                    wait_s((b + NSB - 1) % NSB)

                @pl.when(kf > 1)
                def _():
                    wait_s((b + NSB - 2) % NSB)
        plsc.subcore_barrier()

        pltpu.sync_copy(acc_sh.at[pl.ds(sid * RPT, RPT)],
                        out_hbm.at[pl.ds(win * R + sid * RPT, RPT)])
        plsc.subcore_barrier()
        return carry

    lax.fori_loop(0, WPS, win_pass, 0)


def _scatter(contrib, omap_pad, bias):
    f = pl.kernel(
        _scatter_body,
        out_type=jax.ShapeDtypeStruct((N_PAD, C), jnp.float32),
        mesh=_MESH,
        compiler_params=_PARAMS,
        scratch_types=[
            pltpu.VMEM((OC,), jnp.int32),
            pltpu.VMEM((OC,), jnp.int32),
            pltpu.VMEM((RING,), jnp.int32),
            pltpu.VMEM((C,), jnp.float32),
            pltpu.VMEM((BT, C), jnp.float32),
            pltpu.VMEM((SCH,), jnp.int32),
            pltpu.VMEM((SCH,), jnp.int32),
            pltpu.VMEM((SCH, C), jnp.float32),
            pltpu.VMEM((SCH,), jnp.int32),
            pltpu.VMEM((SCH,), jnp.int32),
            pltpu.VMEM((SCH, C), jnp.float32),
            pltpu.VMEM((SCH,), jnp.int32),
            pltpu.VMEM((SCH,), jnp.int32),
            pltpu.VMEM((SCH, C), jnp.float32),
            pltpu.VMEM_SHARED((R + L, C), jnp.float32),
        ] + [pltpu.SemaphoreType.DMA for _ in range(8)],
    )
    return f(contrib, omap_pad, bias)


def kernel(features, in_map, out_map, weight, bias):
    im = in_map.reshape(-1)
    om = out_map.reshape(-1)
    im_pad = jnp.concatenate(
        [im, jnp.zeros((E_PAD - E,), jnp.int32)])
    om_pad = jnp.concatenate(
        [om, jnp.full((E_PAD - E,), SENTINEL, jnp.int32)])
    gathered = _gather(features, im_pad)
    contrib = _gemm(gathered, weight)
    out_pad = _scatter(contrib, om_pad, bias)
    return out_pad[:N]


# speedup vs baseline: 3.6395x; 1.0474x over previous
import jax
import jax.numpy as jnp
from jax import lax
from jax.experimental import pallas as pl
from jax.experimental.pallas import tpu as pltpu
from jax.experimental.pallas import tpu_sc as plsc

N = 100000
K = 27
M = 20000
C = 128
CI = 64
E = K * M

NC, NS, L = 2, 16, 16
NW = NC * NS

GCH = 128
EPW = 16896
E_PAD = EPW * NW
EPS = E_PAD // NS
GSET = 3
NGRP = EPW // (GCH * GSET)
SCH = 64

R = 12544
WIN = 8
WPS = WIN // NC
N_PAD = R * WIN
RPT = R // NS
OC = 1536
NOC = EPS // OC
BT = 8
NSB = 3
RING = 4 * SCH
SENTINEL = 1 << 30

_MESH = plsc.VectorSubcoreMesh(core_axis_name="c", subcore_axis_name="s")
_PARAMS = pltpu.CompilerParams(needs_layout_passes=False)
_PARAMS_NT = pltpu.CompilerParams(needs_layout_passes=False,
                                  use_tc_tiling_on_sc=False)


def _gather_body(feat_hbm, idx_hbm, out_hbm, idx_v,
                 ra0, ra1, ra2, rb0, rb1, rb2,
                 sem_ga, sem_gb, sem_wa, sem_wb):
    wid = lax.axis_index("s") * NC + lax.axis_index("c")
    base = wid * EPW
    pltpu.sync_copy(idx_hbm.at[pl.ds(base, EPW)], idx_v)

    sets = ((ra0, ra1, ra2, sem_ga, sem_wa), (rb0, rb1, rb2, sem_gb, sem_wb))

    def start_g(g, s):
        r0, r1, r2, sg, _ = sets[s]
        for b, rr in enumerate((r0, r1, r2)):
            ch = g * GSET + b
            pltpu.async_copy(
                feat_hbm.at[idx_v.at[pl.ds(ch * GCH, GCH)]], rr, sg)

    def wait_g(s):
        r0, r1, r2, sg, _ = sets[s]
        for rr in (r0, r1, r2):
            pltpu.make_async_copy(feat_hbm.at[idx_v.at[pl.ds(0, GCH)]],
                                  rr, sg).wait()

    def start_w(g, s):
        r0, r1, r2, _, sw = sets[s]
        for b, rr in enumerate((r0, r1, r2)):
            ch = g * GSET + b
            pltpu.async_copy(rr, out_hbm.at[pl.ds(base + ch * GCH, GCH)], sw)

    def wait_w(s):
        r0, r1, r2, _, sw = sets[s]
        for rr in (r0, r1, r2):
            pltpu.make_async_copy(rr, out_hbm.at[pl.ds(base, GCH)], sw).wait()

    def pair(t, carry):
        for s in range(2):
            g = 2 * t + s

            @pl.when(g >= 2)
            def _():
                wait_w(s)
            start_g(g, s)

            @pl.when(g >= 1)
            def _():
                wait_g(1 - s)
                start_w(g - 1, 1 - s)
        return carry

    lax.fori_loop(0, NGRP // 2, pair, 0)
    wait_g(1)
    start_w(NGRP - 1, 1)
    wait_w(0)
    wait_w(1)


def _gather(features, idx_pad):
    f = pl.kernel(
        _gather_body,
        out_type=jax.ShapeDtypeStruct((E_PAD, C), jnp.float32),
        mesh=_MESH,
        compiler_params=_PARAMS,
        scratch_types=[pltpu.VMEM((EPW,), jnp.int32)]
        + [pltpu.VMEM((GCH, C), jnp.float32) for _ in range(6)]
        + [pltpu.SemaphoreType.DMA for _ in range(4)],
    )
    return f(features, idx_pad)


def _gemm_block(g_ref, w_ref, o_ref):
    o_ref[...] = jnp.dot(g_ref[...], w_ref[0],
                         preferred_element_type=jnp.float32)


def _gemm(gathered, weight):
    BM = 10000
    grid = (K, M // BM)
    return pl.pallas_call(
        _gemm_block,
        grid=grid,
        in_specs=[
            pl.BlockSpec((BM, C), lambda k, mb: (k * (M // BM) + mb, 0)),
            pl.BlockSpec((1, C, C), lambda k, mb: (k, 0, 0)),
        ],
        out_specs=pl.BlockSpec((BM, C), lambda k, mb: (k * (M // BM) + mb, 0)),
        out_shape=jax.ShapeDtypeStruct((E, C), jnp.float32),
    )(gathered, weight)


def _scatter_body(contrib_hbm, omap_hbm, bias_hbm, out_hbm,
                  omap_c0, omap_c1, sel_v, bias_v, bias_t,
                  pos0, row0, rows0, pos1, row1, rows1, pos2, row2, rows2,
                  acc_sh, sem_i, sem_o,
                  sem_g0, sem_g1, sem_g2, sem_s0, sem_s1, sem_s2):
    cid = lax.axis_index("c")
    sid = lax.axis_index("s")
    ebase = sid * EPS

    pltpu.sync_copy(bias_hbm, bias_v)
    for r in range(BT):
        for j in range(C // L):
            bias_t[r, pl.ds(j * L, L)] = bias_v[pl.ds(j * L, L)]

    lane = lax.iota(jnp.int32, L)
    lane14 = lane << jnp.full((L,), 14, jnp.int32)
    maskring = jnp.full((L,), RING - 1, jnp.int32)
    ebase_v = jnp.full((L,), ebase, jnp.int32)
    fourteen = jnp.full((L,), 14, jnp.int32)
    mask14 = jnp.full((L,), 16383, jnp.int32)
    dummy = jnp.full((L,), R, jnp.int32)
    ones = jnp.full((L,), 1, jnp.int32)

    omaps = (omap_c0, omap_c1)
    bufs = ((pos0, row0, rows0, sem_g0, sem_s0),
            (pos1, row1, rows1, sem_g1, sem_s1),
            (pos2, row2, rows2, sem_g2, sem_s2))

    def unpack_start(k, b):
        pos_c, row_c, rows_v, sg, _ = bufs[b]
        boff = (k & 3) * SCH
        for j in range(SCH // L):
            pk = sel_v[pl.ds(boff + j * L, L)]
            pos_c[pl.ds(j * L, L)] = (pk >> fourteen) + ebase_v
            row_c[pl.ds(j * L, L)] = pk & mask14
        pltpu.async_copy(contrib_hbm.at[pos_c], rows_v, sg)

    def wait_g(b):
        pos_c, _, rows_v, sg, _ = bufs[b]
        pltpu.make_async_copy(contrib_hbm.at[pos_c], rows_v, sg).wait()

    def start_s(b):
        _, row_c, rows_v, _, ss = bufs[b]
        pltpu.async_copy(rows_v, acc_sh.at[row_c], ss, add=True)

    def wait_s(b):
        _, row_c, rows_v, _, ss = bufs[b]
        pltpu.make_async_copy(rows_v, acc_sh.at[row_c], ss).wait()

    def flush(cnt):
        k = cnt >> 6
        kb = lax.rem(k, NSB)
        for b in range(NSB):
            @pl.when(kb == b)
            def _():
                @pl.when(k > 0)
                def _():
                    wait_g((b + NSB - 1) % NSB)
                    start_s((b + NSB - 1) % NSB)
                @pl.when(k > 2)
                def _():
                    wait_s(b)
                unpack_start(k, b)

    def append(cnt, packed, m):
        mi = m.astype(jnp.int32)
        ps = plsc.cumsum(mi)
        cnt_v = jnp.full((L,), cnt - 1, jnp.int32)
        plsc.store_scatter(sel_v, [(cnt_v + ps) & maskring], packed, mask=m)
        newcnt = cnt + ps[15]

        @pl.when((newcnt >> 6) > (cnt >> 6))
        def _():
            flush(cnt)
        return newcnt

    def load_omap(ci, which):
        pltpu.async_copy(omap_hbm.at[pl.ds(ebase + ci * OC, OC)],
                         omaps[which], sem_o)

    def wait_omap(which):
        pltpu.make_async_copy(omap_hbm.at[pl.ds(0, OC)],
                              omaps[which], sem_o).wait()

    def win_pass(p, carry):
        win = cid * WPS + p
        lo = win * R
        lo_v = jnp.full((L,), lo, jnp.int32)
        hi_v = jnp.full((L,), lo + R, jnp.int32)

        def init(i, c):
            pltpu.async_copy(bias_t, acc_sh.at[pl.ds(sid * RPT + i * BT, BT)],
                             sem_i)
            return c
        lax.fori_loop(0, RPT // BT, init, 0)
        load_omap(0, 0)

        def init_w(i, c):
            pltpu.make_async_copy(bias_t, acc_sh.at[pl.ds(0, BT)],
                                  sem_i).wait()
            return c
        lax.fori_loop(0, RPT // BT, init_w, 0)
        plsc.subcore_barrier()

        def scan_chunk(ci, cnt, which):
            wait_omap(which)

            @pl.when(ci + 1 < NOC)
            def _():
                load_omap(ci + 1, 1 - which)
            omap_c = omaps[which]

            def scan(i, cnt):
                v = omap_c[pl.ds(i * L, L)]
                m = (v >= lo_v) & (v < hi_v)
                base_v = jnp.full((L,), ci * OC, jnp.int32) + \
                    jnp.full((L,), i * L, jnp.int32)
                packed = ((base_v << fourteen) + lane14) + (v - lo_v)
                return append(cnt, packed, m)
            return lax.fori_loop(0, OC // L, scan, cnt)

        def chunk_pair(t, cnt):
            cnt = scan_chunk(2 * t, cnt, 0)
            cnt = scan_chunk(2 * t + 1, cnt, 1)
            return cnt
        cnt = lax.fori_loop(0, NOC // 2, chunk_pair, 0)

        def pad(i, cnt):
            return append(cnt, dummy, ones > 0)
        cntf = lax.fori_loop(0, SCH // L, pad, cnt)

        kf = (cntf >> 6) - 1
        kfb = lax.rem(kf, NSB)
        for b in range(NSB):
            @pl.when(kfb == b)
            def _():
                wait_g(b)
                start_s(b)
                wait_s(b)

                @pl.when(kf > 0)
                def _():
                    wait_s((b + NSB - 1) % NSB)

                @pl.when(kf > 1)
                def _():
                    wait_s((b + NSB - 2) % NSB)
        plsc.subcore_barrier()

        pltpu.sync_copy(acc_sh.at[pl.ds(sid * RPT, RPT)],
                        out_hbm.at[pl.ds(win * R + sid * RPT, RPT)])
        plsc.subcore_barrier()
        return carry

    lax.fori_loop(0, WPS, win_pass, 0)


def _scatter(contrib, omap_pad, bias):
    f = pl.kernel(
        _scatter_body,
        out_type=jax.ShapeDtypeStruct((N_PAD, C), jnp.float32),
        mesh=_MESH,
        compiler_params=_PARAMS,
        scratch_types=[
            pltpu.VMEM((OC,), jnp.int32),
            pltpu.VMEM((OC,), jnp.int32),
            pltpu.VMEM((RING,), jnp.int32),
            pltpu.VMEM((C,), jnp.float32),
            pltpu.VMEM((BT, C), jnp.float32),
            pltpu.VMEM((SCH,), jnp.int32),
            pltpu.VMEM((SCH,), jnp.int32),
            pltpu.VMEM((SCH, C), jnp.float32),
            pltpu.VMEM((SCH,), jnp.int32),
            pltpu.VMEM((SCH,), jnp.int32),
            pltpu.VMEM((SCH, C), jnp.float32),
            pltpu.VMEM((SCH,), jnp.int32),
            pltpu.VMEM((SCH,), jnp.int32),
            pltpu.VMEM((SCH, C), jnp.float32),
            pltpu.VMEM_SHARED((R + L, C), jnp.float32),
        ] + [pltpu.SemaphoreType.DMA for _ in range(8)],
    )
    return f(contrib, omap_pad, bias)


def kernel(features, in_map, out_map, weight, bias):
    im = in_map.reshape(-1)
    om = out_map.reshape(-1)
    im_pad = jnp.concatenate(
        [im, jnp.zeros((E_PAD - E,), jnp.int32)])
    om_pad = jnp.concatenate(
        [om, jnp.full((E_PAD - E,), SENTINEL, jnp.int32)])
    gathered = _gather(features, im_pad)
    contrib = _gemm(gathered, weight)
    out_pad = _scatter(contrib, om_pad, bias)
    return out_pad[:N]
